# Initial kernel scaffold; baseline (speedup 1.0000x reference)
import jax
import jax.numpy as jnp
from jax.experimental import pallas as pl


def kernel(x):
    raise NotImplementedError("write your pallas kernel here")


# speedup vs baseline: 1.8998x; 1.8998x over previous
import functools

import jax
import jax.numpy as jnp
from jax import lax
from jax.experimental import pallas as pl
from jax.experimental.pallas import tpu as pltpu
from jax.experimental.pallas import tpu_sc as plsc

_START = 128
_F = 256
_HF = _F - _START

_NC = 2
_NS = 16
_NW = _NC * _NS
_L = 16


def _sc_permute(x_flat, idx_flat, n_frames):
    frames_pw = n_frames // _NW
    ch = 128
    n_chunks = frames_pw // ch
    mesh = plsc.VectorSubcoreMesh(core_axis_name="c", subcore_axis_name="s")

    @functools.partial(
        pl.kernel,
        out_type=jax.ShapeDtypeStruct((n_frames * _F,), jnp.float32),
        mesh=mesh,
        compiler_params=pltpu.CompilerParams(needs_layout_passes=False),
        scratch_types=[
            pltpu.VMEM((ch * _F,), jnp.float32),
            pltpu.VMEM((ch * _HF,), jnp.int32),
        ],
    )
    def k(x_hbm, idx_hbm, out_hbm, xv, idxv):
        wid = lax.axis_index("s") * _NC + lax.axis_index("c")
        for c in range(n_chunks):
            frame0 = (wid * n_chunks + c) * ch
            pltpu.sync_copy(x_hbm.at[pl.ds(frame0 * _F, ch * _F)], xv)
            pltpu.sync_copy(idx_hbm.at[pl.ds(frame0 * _HF, ch * _HF)], idxv)

            def body(f, carry):
                fb = f * _F
                gathered = []
                for s in range(_HF // _L):
                    col = idxv[pl.ds(f * _HF + s * _L, _L)]
                    gathered.append(plsc.load_gather(xv, [col + fb]))
                for s in range(_HF // _L):
                    xv[pl.ds(fb + _START + s * _L, _L)] = gathered[s]
                return carry

            lax.fori_loop(0, ch, body, 0)
            pltpu.sync_copy(xv, out_hbm.at[pl.ds(frame0 * _F, ch * _F)])

    return k(x_flat, idx_flat)


def kernel(x):
    B, T, F = x.shape
    r = jax.random.uniform(jax.random.key(1234), (B, T, F - _START),
                           dtype=jnp.float32)
    hf_perm = jnp.argsort(r, axis=-1).astype(jnp.int32) + _START
    out = _sc_permute(x.reshape(-1), hf_perm.reshape(-1), B * T)
    return out.reshape(B, T, F)
